# baseline (speedup 1.0000x reference)
import functools

import jax
import jax.numpy as jnp
from jax import lax
from jax.experimental import pallas as pl
from jax.experimental.pallas import tpu as pltpu
from jax.experimental.pallas import tpu_sc as plsc

N, H, W, C = 8, 224, 224, 96
NC, NS = 2, 16
NW = NC * NS
ROWS_PER_N = H // NW
HALF = W // 2
NCHUNK = HALF // 16
CBLK = C // 16
CP = 128

_SCALE = 2.0 / (W - 1)


def _bf16_round(x):
    u = lax.bitcast_convert_type(x, jnp.int32)
    lsb = lax.shift_right_logical(u, 16) & 1
    r = (u + 32767 + lsb) & jnp.int32(-65536)
    return lax.bitcast_convert_type(r, jnp.float32)


def _floor_clip(x):
    t = x.astype(jnp.int32)
    tf = t.astype(jnp.float32)
    fl = jnp.where(tf > x, t - 1, t)
    i0 = jnp.minimum(jnp.maximum(fl, 0), W - 1)
    i1 = jnp.minimum(jnp.maximum(fl + 1, 0), W - 1)
    return i0, i1, i0.astype(jnp.float32), i1.astype(jnp.float32)


def _body(tbl, theta_hbm, out_hbm,
          theta_v, ia_v, ib_v, ic_v, id_v, u_v, up_v, v_v, vp_v,
          bufa, bufb, bufc, bufd, out_v, sem):
    wid = lax.axis_index("s") * NC + lax.axis_index("c")

    pltpu.sync_copy(theta_hbm, theta_v)

    iota = lax.iota(jnp.int32, 16)

    def sample_body(n, _):
        base_row = n * (H * W)
        th = _bf16_round(theta_v[pl.ds(n * 6, 16)])
        a00 = th[0]
        a01 = th[1]
        a02 = th[2]
        a10 = th[3]
        a11 = th[4]
        a12 = th[5]

        def half_body(t, _):
            i = wid * ROWS_PER_N + (t >> 1)
            jb = (t & 1) * HALF
            yt = _bf16_round(
                (iota * 0 + i).astype(jnp.float32) * _SCALE - 1.0)

            for k in range(NCHUNK):
                sl = pl.ds(k * 16, 16)
                jv = jb + k * 16 + iota
                xt = _bf16_round(jv.astype(jnp.float32) * _SCALE - 1.0)
                xs = a00 * xt + a01 * yt + a02
                ys = a10 * xt + a11 * yt + a12
                xv = (xs + 1.0) * (W / 2)
                yv = (ys + 1.0) * (H / 2)
                x0, x1, x0f, x1f = _floor_clip(xv)
                y0, y1, y0f, y1f = _floor_clip(yv)
                ia_v[sl] = base_row + y0 * W + x0
                ib_v[sl] = base_row + y1 * W + x0
                ic_v[sl] = base_row + y0 * W + x1
                id_v[sl] = base_row + y1 * W + x1
                u_v[sl] = x1f - xv
                up_v[sl] = xv - x0f
                v_v[sl] = y1f - yv
                vp_v[sl] = yv - y0f

            ca = pltpu.async_copy(tbl.at[ia_v], bufa, sem)
            cb = pltpu.async_copy(tbl.at[ib_v], bufb, sem)
            cc = pltpu.async_copy(tbl.at[ic_v], bufc, sem)
            cd = pltpu.async_copy(tbl.at[id_v], bufd, sem)
            ca.wait()
            cb.wait()
            cc.wait()
            cd.wait()

            def pix_chunk(q, _):
                pb = q * 16
                uu16 = u_v[pl.ds(pb, 16)]
                uup16 = up_v[pl.ds(pb, 16)]
                vv16 = v_v[pl.ds(pb, 16)]
                vvp16 = vp_v[pl.ds(pb, 16)]
                for l in range(16):
                    p = pb + l
                    uu = uu16[l]
                    uup = uup16[l]
                    vv = vv16[l]
                    vvp = vvp16[l]
                    for c in range(CBLK):
                        cs = pl.ds(c * 16, 16)
                        sa = bufa[p, cs]
                        sb = bufb[p, cs]
                        sc = bufc[p, cs]
                        sd = bufd[p, cs]
                        m1 = vv * sa + vvp * sb
                        m2 = vv * sc + vvp * sd
                        out_v[p, cs] = uu * m1 + uup * m2
                return 0

            lax.fori_loop(0, NCHUNK, pix_chunk, 0)

            dst = base_row + i * W + jb
            pltpu.sync_copy(out_v, out_hbm.at[pl.ds(dst, HALF)])
            return 0

        lax.fori_loop(0, 2 * ROWS_PER_N, half_body, 0)
        return 0

    lax.fori_loop(0, N, sample_body, 0)


@jax.jit
def _run(tbl, theta_pad):
    mesh = plsc.VectorSubcoreMesh(core_axis_name="c", subcore_axis_name="s")
    f = pl.kernel(
        _body,
        out_type=jax.ShapeDtypeStruct((N * H * W, C), jnp.float32),
        mesh=mesh,
        scratch_types=[
            pltpu.VMEM((64,), jnp.float32),
            pltpu.VMEM((HALF,), jnp.int32),
            pltpu.VMEM((HALF,), jnp.int32),
            pltpu.VMEM((HALF,), jnp.int32),
            pltpu.VMEM((HALF,), jnp.int32),
            pltpu.VMEM((HALF,), jnp.float32),
            pltpu.VMEM((HALF,), jnp.float32),
            pltpu.VMEM((HALF,), jnp.float32),
            pltpu.VMEM((HALF,), jnp.float32),
            pltpu.VMEM((HALF, CP), jnp.float32),
            pltpu.VMEM((HALF, CP), jnp.float32),
            pltpu.VMEM((HALF, CP), jnp.float32),
            pltpu.VMEM((HALF, CP), jnp.float32),
            pltpu.VMEM((HALF, C), jnp.float32),
            pltpu.SemaphoreType.DMA,
        ],
    )
    return f(tbl, theta_pad)


_PAD_ROWS = 3584


def _pad_block(x_ref, o_ref):
    o_ref[...] = jnp.concatenate(
        [x_ref[...], jnp.zeros((_PAD_ROWS, CP - C), jnp.float32)], axis=1)


def _pad_rows(x2d):
    return pl.pallas_call(
        _pad_block,
        grid=(N * H * W // _PAD_ROWS,),
        in_specs=[pl.BlockSpec((_PAD_ROWS, C), lambda g: (g, 0))],
        out_specs=pl.BlockSpec((_PAD_ROWS, CP), lambda g: (g, 0)),
        out_shape=jax.ShapeDtypeStruct((N * H * W, CP), jnp.float32),
    )(x2d)


def kernel(X, theta):
    tbl = _pad_rows(X.reshape(N * H * W, C))
    theta_pad = jnp.concatenate(
        [theta.reshape(-1), jnp.zeros(16, jnp.float32)])
    out = _run(tbl, theta_pad)
    return out.reshape(N, H, W, C)


# speedup vs baseline: 1.5084x; 1.5084x over previous
import functools

import jax
import jax.numpy as jnp
from jax import lax
from jax.experimental import pallas as pl
from jax.experimental.pallas import tpu as pltpu
from jax.experimental.pallas import tpu_sc as plsc

N, H, W, C = 8, 224, 224, 96
NC, NS = 2, 16
NW = NC * NS
ROWS_PER_N = H // NW
HALF = W // 2
NCHUNK = HALF // 16
CBLK = C // 16
CP = 128

_SCALE = 2.0 / (W - 1)


def _bf16_round(x):
    u = lax.bitcast_convert_type(x, jnp.int32)
    lsb = lax.shift_right_logical(u, 16) & 1
    r = (u + 32767 + lsb) & jnp.int32(-65536)
    return lax.bitcast_convert_type(r, jnp.float32)


def _floor_clip(x):
    t = x.astype(jnp.int32)
    tf = t.astype(jnp.float32)
    fl = jnp.where(tf > x, t - 1, t)
    i0 = jnp.minimum(jnp.maximum(fl, 0), W - 1)
    i1 = jnp.minimum(jnp.maximum(fl + 1, 0), W - 1)
    return i0, i1, i0.astype(jnp.float32), i1.astype(jnp.float32)


def _body(tbl, theta_hbm, out_hbm,
          theta_v, ia_v, ib_v, ic_v, id_v, u_v, up_v, v_v, vp_v,
          bufa, bufb, bufc, bufd, out_v, sem):
    wid = lax.axis_index("s") * NC + lax.axis_index("c")

    pltpu.sync_copy(theta_hbm, theta_v)

    iota = lax.iota(jnp.int32, 16)

    def sample_body(n, _):
        base_row = n * (H * W)
        th = _bf16_round(theta_v[pl.ds(n * 6, 16)])
        a00 = th[0]
        a01 = th[1]
        a02 = th[2]
        a10 = th[3]
        a11 = th[4]
        a12 = th[5]

        def half_body(t, _):
            i = wid * ROWS_PER_N + (t >> 1)
            jb = (t & 1) * HALF
            yt = _bf16_round(
                (iota * 0 + i).astype(jnp.float32) * _SCALE - 1.0)

            for k in range(NCHUNK):
                sl = pl.ds(k * 16, 16)
                jv = jb + k * 16 + iota
                xt = _bf16_round(jv.astype(jnp.float32) * _SCALE - 1.0)
                xs = a00 * xt + a01 * yt + a02
                ys = a10 * xt + a11 * yt + a12
                xv = (xs + 1.0) * (W / 2)
                yv = (ys + 1.0) * (H / 2)
                x0, x1, x0f, x1f = _floor_clip(xv)
                y0, y1, y0f, y1f = _floor_clip(yv)
                ia_v[sl] = base_row + y0 * W + x0
                ib_v[sl] = base_row + y1 * W + x0
                ic_v[sl] = base_row + y0 * W + x1
                id_v[sl] = base_row + y1 * W + x1
                u_v[sl] = x1f - xv
                up_v[sl] = xv - x0f
                v_v[sl] = y1f - yv
                vp_v[sl] = yv - y0f

            ca = pltpu.async_copy(tbl.at[ia_v], bufa, sem)
            cb = pltpu.async_copy(tbl.at[ib_v], bufb, sem)
            cc = pltpu.async_copy(tbl.at[ic_v], bufc, sem)
            cd = pltpu.async_copy(tbl.at[id_v], bufd, sem)
            ca.wait()
            cb.wait()
            cc.wait()
            cd.wait()

            def pix_chunk(q, _):
                pb = q * 16
                uu16 = u_v[pl.ds(pb, 16)]
                uup16 = up_v[pl.ds(pb, 16)]
                vv16 = v_v[pl.ds(pb, 16)]
                vvp16 = vp_v[pl.ds(pb, 16)]
                for l in range(16):
                    p = pb + l
                    uu = uu16[l]
                    uup = uup16[l]
                    vv = vv16[l]
                    vvp = vvp16[l]
                    for c in range(CBLK):
                        cs = pl.ds(c * 16, 16)
                        sa = bufa[p, cs]
                        sb = bufb[p, cs]
                        sc = bufc[p, cs]
                        sd = bufd[p, cs]
                        m1 = vv * sa + vvp * sb
                        m2 = vv * sc + vvp * sd
                        out_v[p, cs] = uu * m1 + uup * m2
                return 0

            lax.fori_loop(0, NCHUNK, pix_chunk, 0)

            dst = base_row + i * W + jb
            pltpu.sync_copy(out_v, out_hbm.at[pl.ds(dst, HALF)])
            return 0

        lax.fori_loop(0, 2 * ROWS_PER_N, half_body, 0)
        return 0

    lax.fori_loop(0, N, sample_body, 0)


@jax.jit
def _run(tbl, theta_pad):
    mesh = plsc.VectorSubcoreMesh(core_axis_name="c", subcore_axis_name="s")
    f = pl.kernel(
        _body,
        out_type=jax.ShapeDtypeStruct((N * H * W, C), jnp.float32),
        mesh=mesh,
        scratch_types=[
            pltpu.VMEM((64,), jnp.float32),
            pltpu.VMEM((HALF,), jnp.int32),
            pltpu.VMEM((HALF,), jnp.int32),
            pltpu.VMEM((HALF,), jnp.int32),
            pltpu.VMEM((HALF,), jnp.int32),
            pltpu.VMEM((HALF,), jnp.float32),
            pltpu.VMEM((HALF,), jnp.float32),
            pltpu.VMEM((HALF,), jnp.float32),
            pltpu.VMEM((HALF,), jnp.float32),
            pltpu.VMEM((HALF, CP), jnp.float32),
            pltpu.VMEM((HALF, CP), jnp.float32),
            pltpu.VMEM((HALF, CP), jnp.float32),
            pltpu.VMEM((HALF, CP), jnp.float32),
            pltpu.VMEM((HALF, C), jnp.float32),
            pltpu.SemaphoreType.DMA,
        ],
    )
    return f(tbl, theta_pad)


_PLANES_PER_BLK = 8


def _tab_block(x_ref, o_ref):
    t = jnp.transpose(x_ref[...], (0, 2, 1))
    o_ref[...] = jnp.concatenate(
        [t, jnp.zeros((_PLANES_PER_BLK, W, CP - C), jnp.float32)], axis=2)


def _build_table(xplanar):
    PL = _PLANES_PER_BLK
    tab = pl.pallas_call(
        _tab_block,
        grid=(N * H // PL,),
        in_specs=[pl.BlockSpec((PL, C, W), lambda g: (g, 0, 0))],
        out_specs=pl.BlockSpec((PL, W, CP), lambda g: (g, 0, 0)),
        out_shape=jax.ShapeDtypeStruct((N * H, W, CP), jnp.float32),
    )(xplanar)
    return tab.reshape(N * H * W, CP)


def kernel(X, theta):
    xplanar = jnp.transpose(X, (0, 1, 3, 2)).reshape(N * H, C, W)
    tbl = _build_table(xplanar)
    theta_pad = jnp.concatenate(
        [theta.reshape(-1), jnp.zeros(16, jnp.float32)])
    out = _run(tbl, theta_pad)
    return out.reshape(N, H, W, C)


# speedup vs baseline: 1.7530x; 1.1621x over previous
import functools

import jax
import jax.numpy as jnp
from jax import lax
from jax.experimental import pallas as pl
from jax.experimental.pallas import tpu as pltpu
from jax.experimental.pallas import tpu_sc as plsc

N, H, W, C = 8, 224, 224, 96
NC, NS = 2, 16
NW = NC * NS
ROWS_PER_N = H // NW
PIECE = 64
NPIECE = 4
NCHUNK = PIECE // 16
NPAIR = ROWS_PER_N * NPIECE // 2
CBLK = C // 16
CP = 128

_SCALE = 2.0 / (W - 1)


def _bf16_round(x):
    u = lax.bitcast_convert_type(x, jnp.int32)
    lsb = lax.shift_right_logical(u, 16) & 1
    r = (u + 32767 + lsb) & jnp.int32(-65536)
    return lax.bitcast_convert_type(r, jnp.float32)


def _floor_clip(x):
    t = x.astype(jnp.int32)
    tf = t.astype(jnp.float32)
    fl = jnp.where(tf > x, t - 1, t)
    i0 = jnp.minimum(jnp.maximum(fl, 0), W - 1)
    i1 = jnp.minimum(jnp.maximum(fl + 1, 0), W - 1)
    return i0, i1, i0.astype(jnp.float32), i1.astype(jnp.float32)


def _body(tbl, theta_hbm, out_hbm,
          theta_v,
          ia0, ib0, ic0, id0, ia1, ib1, ic1, id1,
          u0, v0, u1, v1,
          a0, b0, c0, d0, a1, b1, c1, d1,
          out_v, sg0, sg1, so):
    wid = lax.axis_index("s") * NC + lax.axis_index("c")

    pltpu.sync_copy(theta_hbm, theta_v)

    iota = lax.iota(jnp.int32, 16)

    IDX = ((ia0, ib0, ic0, id0), (ia1, ib1, ic1, id1))
    WT = ((u0, v0), (u1, v1))
    BUF = ((a0, b0, c0, d0), (a1, b1, c1, d1))
    SG = (sg0, sg1)

    def chunk_coords(th, yt, jb, pb):
        a00, a01, a02 = th[0], th[1], th[2]
        a10, a11, a12 = th[3], th[4], th[5]
        jv = jb + pb + iota
        xt = _bf16_round(jv.astype(jnp.float32) * _SCALE - 1.0)
        xs = a00 * xt + a01 * yt + a02
        ys = a10 * xt + a11 * yt + a12
        xv = (xs + 1.0) * (W / 2)
        yv = (ys + 1.0) * (H / 2)
        return xv, yv

    def row_consts(t):
        i = wid * ROWS_PER_N + (t >> 2)
        piece = t & 3
        jb = jnp.where(piece == 3, W - PIECE, piece * PIECE)
        yt = _bf16_round((iota * 0 + i).astype(jnp.float32) * _SCALE - 1.0)
        return i, jb, yt

    def coords_fire(th, base_row, t, P):
        ia_v, ib_v, ic_v, id_v = IDX[P]
        u_v, v_v = WT[P]
        i, jb, yt = row_consts(t)
        for k in range(NCHUNK):
            sl = pl.ds(k * 16, 16)
            xv, yv = chunk_coords(th, yt, jb, k * 16)
            x0, x1, x0f, x1f = _floor_clip(xv)
            y0, y1, y0f, y1f = _floor_clip(yv)
            ia_v[sl] = base_row + y0 * W + x0
            ib_v[sl] = base_row + y1 * W + x0
            ic_v[sl] = base_row + y0 * W + x1
            id_v[sl] = base_row + y1 * W + x1
            u_v[sl] = x1f - xv
            v_v[sl] = y1f - yv
        for x in range(4):
            pltpu.async_copy(tbl.at[IDX[P][x]], BUF[P][x], SG[P])

    def drain_out():
        pltpu.make_async_copy(out_v, out_hbm.at[pl.ds(0, PIECE)], so).wait()

    def combine(th, base_row, t, P, first):
        for x in range(4):
            pltpu.make_async_copy(tbl.at[IDX[P][x]], BUF[P][x],
                                  SG[P]).wait()
        if first is None:
            drain_out()
        else:
            @pl.when(jnp.logical_not(first))
            def _():
                drain_out()
        bufa, bufb, bufc, bufd = BUF[P]
        ia_v, ib_v, ic_v, _idv = IDX[P]
        u_v, v_v = WT[P]
        i, jb, yt = row_consts(t)

        def pix_chunk(q, _):
            pb = q * 16
            sl = pl.ds(pb, 16)
            ia16 = ia_v[sl]
            ib16 = ib_v[sl]
            ic16 = ic_v[sl]
            uu16 = u_v[sl]
            vv16 = v_v[sl]
            uup16 = (ic16 - ia16).astype(jnp.float32) - uu16
            vvp16 = jnp.where(ib16 > ia16, 1.0, 0.0) - vv16
            for l in range(16):
                p = pb + l
                uu = uu16[l]
                uup = uup16[l]
                vv = vv16[l]
                vvp = vvp16[l]
                for c in range(CBLK):
                    cs = pl.ds(c * 16, 16)
                    sa = bufa[p, cs]
                    sb = bufb[p, cs]
                    sc = bufc[p, cs]
                    sd = bufd[p, cs]
                    m1 = vv * sa + vvp * sb
                    m2 = vv * sc + vvp * sd
                    out_v[p, cs] = uu * m1 + uup * m2
            return 0

        lax.fori_loop(0, NCHUNK, pix_chunk, 0)

        dst = base_row + i * W + jb
        pltpu.async_copy(out_v, out_hbm.at[pl.ds(dst, PIECE)], so)

    def sample_body(n, _):
        base_row = n * (H * W)
        th = _bf16_round(theta_v[pl.ds(n * 6, 16)])
        coords_fire(th, base_row, 0, 0)

        def pair(s, _):
            coords_fire(th, base_row, 2 * s + 1, 1)
            combine(th, base_row, 2 * s, 0,
                    first=jnp.logical_and(n == 0, s == 0))

            @pl.when(s < NPAIR - 1)
            def _():
                coords_fire(th, base_row, 2 * s + 2, 0)

            combine(th, base_row, 2 * s + 1, 1, first=None)
            return 0

        lax.fori_loop(0, NPAIR, pair, 0)
        return 0

    lax.fori_loop(0, N, sample_body, 0)
    drain_out()


@jax.jit
def _run(tbl, theta_pad):
    mesh = plsc.VectorSubcoreMesh(core_axis_name="c", subcore_axis_name="s")
    f = pl.kernel(
        _body,
        out_type=jax.ShapeDtypeStruct((N * H * W, C), jnp.float32),
        mesh=mesh,
        scratch_types=(
            [pltpu.VMEM((64,), jnp.float32)]
            + [pltpu.VMEM((PIECE,), jnp.int32)] * 8
            + [pltpu.VMEM((PIECE,), jnp.float32)] * 4
            + [pltpu.VMEM((PIECE, CP), jnp.float32)] * 8
            + [pltpu.VMEM((PIECE, C), jnp.float32)]
            + [pltpu.SemaphoreType.DMA] * 3
        ),
    )
    return f(tbl, theta_pad)


_PLANES_PER_BLK = 8


def _tab_block(x_ref, o_ref):
    t = jnp.transpose(x_ref[...], (0, 2, 1))
    o_ref[...] = jnp.concatenate(
        [t, jnp.zeros((_PLANES_PER_BLK, W, CP - C), jnp.float32)], axis=2)


def _build_table(xplanar):
    PL = _PLANES_PER_BLK
    tab = pl.pallas_call(
        _tab_block,
        grid=(N * H // PL,),
        in_specs=[pl.BlockSpec((PL, C, W), lambda g: (g, 0, 0))],
        out_specs=pl.BlockSpec((PL, W, CP), lambda g: (g, 0, 0)),
        out_shape=jax.ShapeDtypeStruct((N * H, W, CP), jnp.float32),
    )(xplanar)
    return tab.reshape(N * H * W, CP)


def kernel(X, theta):
    xplanar = jnp.transpose(X, (0, 1, 3, 2)).reshape(N * H, C, W)
    tbl = _build_table(xplanar)
    theta_pad = jnp.concatenate(
        [theta.reshape(-1), jnp.zeros(16, jnp.float32)])
    out = _run(tbl, theta_pad)
    return out.reshape(N, H, W, C)


# speedup vs baseline: 1.8751x; 1.0697x over previous
import functools

import jax
import jax.numpy as jnp
from jax import lax
from jax.experimental import pallas as pl
from jax.experimental.pallas import tpu as pltpu
from jax.experimental.pallas import tpu_sc as plsc

N, H, W, C = 8, 224, 224, 96
NC, NS = 2, 16
NW = NC * NS
ROWS_PER_N = H // NW
PIECE = 64
NPIECE = 4
NCHUNK = PIECE // 16
NPAIR = ROWS_PER_N * NPIECE // 2
CBLK = C // 16
CP = 128

_SCALE = 2.0 / (W - 1)


def _bf16_round(x):
    u = lax.bitcast_convert_type(x, jnp.int32)
    lsb = lax.shift_right_logical(u, 16) & 1
    r = (u + 32767 + lsb) & jnp.int32(-65536)
    return lax.bitcast_convert_type(r, jnp.float32)


def _floor_clip(x):
    t = x.astype(jnp.int32)
    tf = t.astype(jnp.float32)
    fl = jnp.where(tf > x, t - 1, t)
    i0 = jnp.minimum(jnp.maximum(fl, 0), W - 1)
    i1 = jnp.minimum(jnp.maximum(fl + 1, 0), W - 1)
    return i0, i1, i0.astype(jnp.float32), i1.astype(jnp.float32)


def _body(tbl, theta_hbm, out_hbm,
          theta_v,
          ia0, ib0, ic0, id0, ia1, ib1, ic1, id1,
          u0, v0, u1, v1,
          a0, b0, c0, d0, a1, b1, c1, d1,
          out_v, sg0, sg1, so):
    wid = lax.axis_index("s") * NC + lax.axis_index("c")

    pltpu.sync_copy(theta_hbm, theta_v)

    iota = lax.iota(jnp.int32, 16)

    IDX = ((ia0, ib0, ic0, id0), (ia1, ib1, ic1, id1))
    WT = ((u0, v0), (u1, v1))
    BUF = ((a0, b0, c0, d0), (a1, b1, c1, d1))
    SG = (sg0, sg1)

    def chunk_coords(th, yt, jb, pb):
        a00, a01, a02 = th[0], th[1], th[2]
        a10, a11, a12 = th[3], th[4], th[5]
        jv = jb + pb + iota
        xt = _bf16_round(jv.astype(jnp.float32) * _SCALE - 1.0)
        xs = a00 * xt + a01 * yt + a02
        ys = a10 * xt + a11 * yt + a12
        xv = (xs + 1.0) * (W / 2)
        yv = (ys + 1.0) * (H / 2)
        return xv, yv

    def row_consts(t):
        i = wid * ROWS_PER_N + (t >> 2)
        piece = t & 3
        jb = jnp.where(piece == 3, W - PIECE, piece * PIECE)
        yt = _bf16_round((iota * 0 + i).astype(jnp.float32) * _SCALE - 1.0)
        return i, jb, yt

    def coords_fire(th, base_row, t, P):
        ia_v, ib_v, ic_v, id_v = IDX[P]
        u_v, v_v = WT[P]
        i, jb, yt = row_consts(t)
        for k in range(NCHUNK):
            sl = pl.ds(k * 16, 16)
            xv, yv = chunk_coords(th, yt, jb, k * 16)
            x0, x1, x0f, x1f = _floor_clip(xv)
            y0, y1, y0f, y1f = _floor_clip(yv)
            ia_v[sl] = base_row + y0 * W + x0
            ib_v[sl] = base_row + y1 * W + x0
            ic_v[sl] = base_row + y0 * W + x1
            id_v[sl] = base_row + y1 * W + x1
            u_v[sl] = x1f - xv
            v_v[sl] = y1f - yv
        for x in range(4):
            pltpu.async_copy(tbl.at[IDX[P][x]], BUF[P][x], SG[P])

    def drain_out():
        pltpu.make_async_copy(out_v, out_hbm.at[pl.ds(0, PIECE)], so).wait()

    def combine(th, base_row, t, P, first):
        for x in range(4):
            pltpu.make_async_copy(tbl.at[IDX[P][x]], BUF[P][x],
                                  SG[P]).wait()
        if first is None:
            drain_out()
        else:
            @pl.when(jnp.logical_not(first))
            def _():
                drain_out()
        bufa, bufb, bufc, bufd = BUF[P]
        ia_v, ib_v, ic_v, _idv = IDX[P]
        u_v, v_v = WT[P]
        i, jb, yt = row_consts(t)

        def pix_chunk(q, _):
            pb = q * 16
            sl = pl.ds(pb, 16)
            ia16 = ia_v[sl]
            ib16 = ib_v[sl]
            ic16 = ic_v[sl]
            uu16 = u_v[sl]
            vv16 = v_v[sl]
            uup16 = (ic16 - ia16).astype(jnp.float32) - uu16
            vvp16 = jnp.where(ib16 > ia16, 1.0, 0.0) - vv16
            for l in range(16):
                p = pb + l
                uu = uu16[l]
                uup = uup16[l]
                vv = vv16[l]
                vvp = vvp16[l]
                for c in range(CBLK):
                    cs = pl.ds(c * 16, 16)
                    sa = bufa[p, cs]
                    sb = bufb[p, cs]
                    sc = bufc[p, cs]
                    sd = bufd[p, cs]
                    m1 = vv * sa + vvp * sb
                    m2 = vv * sc + vvp * sd
                    out_v[p, cs] = uu * m1 + uup * m2
            return 0

        lax.fori_loop(0, NCHUNK, pix_chunk, 0)

        dst = base_row + i * W + jb
        pltpu.async_copy(out_v, out_hbm.at[pl.ds(dst, PIECE)], so)

    def sample_body(n, _):
        base_row = n * (H * W)
        th = _bf16_round(theta_v[pl.ds(n * 6, 16)])
        coords_fire(th, base_row, 0, 0)

        def pair(s, _):
            coords_fire(th, base_row, 2 * s + 1, 1)
            combine(th, base_row, 2 * s, 0,
                    first=jnp.logical_and(n == 0, s == 0))

            @pl.when(s < NPAIR - 1)
            def _():
                coords_fire(th, base_row, 2 * s + 2, 0)

            combine(th, base_row, 2 * s + 1, 1, first=None)
            return 0

        lax.fori_loop(0, NPAIR, pair, 0)
        return 0

    lax.fori_loop(0, N, sample_body, 0)
    drain_out()


@jax.jit
def _run(tbl, theta_pad):
    mesh = plsc.VectorSubcoreMesh(core_axis_name="c", subcore_axis_name="s")
    f = pl.kernel(
        _body,
        out_type=jax.ShapeDtypeStruct((N * H * W, C), jnp.float32),
        mesh=mesh,
        scratch_types=(
            [pltpu.VMEM((64,), jnp.float32)]
            + [pltpu.VMEM((PIECE,), jnp.int32)] * 8
            + [pltpu.VMEM((PIECE,), jnp.float32)] * 4
            + [pltpu.VMEM((PIECE, CP), jnp.float32)] * 8
            + [pltpu.VMEM((PIECE, C), jnp.float32)]
            + [pltpu.SemaphoreType.DMA] * 3
        ),
    )
    return f(tbl, theta_pad)


_PLANES_PER_BLK = 16


def _tab_block(x_ref, o_ref):
    t = jnp.transpose(x_ref[...], (0, 2, 1))
    o_ref[...] = jnp.concatenate(
        [t, jnp.zeros((_PLANES_PER_BLK, W, CP - C), jnp.float32)], axis=2)


def _build_table(xplanar):
    PL = _PLANES_PER_BLK
    tab = pl.pallas_call(
        _tab_block,
        grid=(N * H // PL,),
        in_specs=[pl.BlockSpec((PL, C, W), lambda g: (g, 0, 0))],
        out_specs=pl.BlockSpec((PL, W, CP), lambda g: (g, 0, 0)),
        out_shape=jax.ShapeDtypeStruct((N * H, W, CP), jnp.float32),
    )(xplanar)
    return tab.reshape(N * H * W, CP)


def kernel(X, theta):
    xplanar = jnp.transpose(X, (0, 1, 3, 2)).reshape(N * H, C, W)
    tbl = _build_table(xplanar)
    theta_pad = jnp.concatenate(
        [theta.reshape(-1), jnp.zeros(16, jnp.float32)])
    out = _run(tbl, theta_pad)
    return out.reshape(N, H, W, C)


# speedup vs baseline: 1.9443x; 1.0369x over previous
import functools

import jax
import jax.numpy as jnp
from jax import lax
from jax.experimental import pallas as pl
from jax.experimental.pallas import tpu as pltpu
from jax.experimental.pallas import tpu_sc as plsc

N, H, W, C = 8, 224, 224, 96
NC, NS = 2, 16
NW = NC * NS
ROWS_PER_N = H // NW
PIECE = 64
NPIECE = 4
NCHUNK = PIECE // 16
NPAIR = ROWS_PER_N * NPIECE // 2
CBLK = C // 16
CP = 128

_SCALE = 2.0 / (W - 1)


def _bf16_round(x):
    u = lax.bitcast_convert_type(x, jnp.int32)
    lsb = lax.shift_right_logical(u, 16) & 1
    r = (u + 32767 + lsb) & jnp.int32(-65536)
    return lax.bitcast_convert_type(r, jnp.float32)


def _floor_clip(x):
    t = x.astype(jnp.int32)
    tf = t.astype(jnp.float32)
    fl = jnp.where(tf > x, t - 1, t)
    i0 = jnp.minimum(jnp.maximum(fl, 0), W - 1)
    i1 = jnp.minimum(jnp.maximum(fl + 1, 0), W - 1)
    return i0, i1, i0.astype(jnp.float32), i1.astype(jnp.float32)


def _body(tbl, theta_hbm, out_hbm,
          theta_v,
          ia0, ib0, ic0, id0, ia1, ib1, ic1, id1,
          u0, v0, u1, v1,
          a0, b0, c0, d0, a1, b1, c1, d1,
          out_v, sg0, sg1, so):
    wid = lax.axis_index("s") * NC + lax.axis_index("c")

    pltpu.sync_copy(theta_hbm, theta_v)

    iota = lax.iota(jnp.int32, 16)

    IDX = ((ia0, ib0, ic0, id0), (ia1, ib1, ic1, id1))
    WT = ((u0, v0), (u1, v1))
    BUF = ((a0, b0, c0, d0), (a1, b1, c1, d1))
    SG = (sg0, sg1)

    def chunk_coords(th, yt, jb, pb):
        a00, a01, a02 = th[0], th[1], th[2]
        a10, a11, a12 = th[3], th[4], th[5]
        jv = jb + pb + iota
        xt = _bf16_round(jv.astype(jnp.float32) * _SCALE - 1.0)
        xs = a00 * xt + a01 * yt + a02
        ys = a10 * xt + a11 * yt + a12
        xv = (xs + 1.0) * (W / 2)
        yv = (ys + 1.0) * (H / 2)
        return xv, yv

    def row_consts(t):
        i = wid * ROWS_PER_N + (t >> 2)
        piece = t & 3
        jb = jnp.where(piece == 3, W - PIECE, piece * PIECE)
        yt = _bf16_round((iota * 0 + i).astype(jnp.float32) * _SCALE - 1.0)
        return i, jb, yt

    def coords_fire(th, base_row, t, P):
        ia_v, ib_v, ic_v, id_v = IDX[P]
        u_v, v_v = WT[P]
        i, jb, yt = row_consts(t)
        for k in range(NCHUNK):
            sl = pl.ds(k * 16, 16)
            xv, yv = chunk_coords(th, yt, jb, k * 16)
            x0, x1, x0f, x1f = _floor_clip(xv)
            y0, y1, y0f, y1f = _floor_clip(yv)
            ia_v[sl] = base_row + y0 * W + x0
            ib_v[sl] = base_row + y1 * W + x0
            ic_v[sl] = base_row + y0 * W + x1
            id_v[sl] = base_row + y1 * W + x1
            u_v[sl] = x1f - xv
            v_v[sl] = y1f - yv
        for x in range(4):
            pltpu.async_copy(tbl.at[IDX[P][x]], BUF[P][x], SG[P])

    def drain_out():
        pltpu.make_async_copy(out_v, out_hbm.at[pl.ds(0, PIECE)], so).wait()

    def combine(th, base_row, t, P, first):
        for x in range(4):
            pltpu.make_async_copy(tbl.at[IDX[P][x]], BUF[P][x],
                                  SG[P]).wait()
        if first is None:
            drain_out()
        else:
            @pl.when(jnp.logical_not(first))
            def _():
                drain_out()
        bufa, bufb, bufc, bufd = BUF[P]
        ia_v, ib_v, ic_v, _idv = IDX[P]
        u_v, v_v = WT[P]
        i, jb, yt = row_consts(t)

        def pix_chunk(q, _):
            pb = q * 16
            sl = pl.ds(pb, 16)
            ia16 = ia_v[sl]
            ib16 = ib_v[sl]
            ic16 = ic_v[sl]
            uu16 = u_v[sl]
            vv16 = v_v[sl]
            uup16 = (ic16 - ia16).astype(jnp.float32) - uu16
            vvp16 = jnp.where(ib16 > ia16, 1.0, 0.0) - vv16
            for l in range(16):
                p = pb + l
                uu = uu16[l]
                uup = uup16[l]
                vv = vv16[l]
                vvp = vvp16[l]
                for c in range(CBLK):
                    cs = pl.ds(c * 16, 16)
                    sa = bufa[p, cs]
                    sb = bufb[p, cs]
                    sc = bufc[p, cs]
                    sd = bufd[p, cs]
                    m1 = vv * sa + vvp * sb
                    m2 = vv * sc + vvp * sd
                    out_v[p, cs] = uu * m1 + uup * m2
            return 0

        lax.fori_loop(0, NCHUNK, pix_chunk, 0)

        dst = base_row + i * W + jb
        pltpu.async_copy(out_v, out_hbm.at[pl.ds(dst, PIECE)], so)

    def sample_body(n, _):
        base_row = n * (H * W)
        th = _bf16_round(theta_v[pl.ds(n * 6, 16)])
        coords_fire(th, base_row, 0, 0)

        def pair(s, _):
            coords_fire(th, base_row, 2 * s + 1, 1)
            combine(th, base_row, 2 * s, 0,
                    first=jnp.logical_and(n == 0, s == 0))

            @pl.when(s < NPAIR - 1)
            def _():
                coords_fire(th, base_row, 2 * s + 2, 0)

            combine(th, base_row, 2 * s + 1, 1, first=None)
            return 0

        lax.fori_loop(0, NPAIR, pair, 0)
        return 0

    lax.fori_loop(0, N, sample_body, 0)
    drain_out()


@jax.jit
def _run(tbl, theta_pad):
    mesh = plsc.VectorSubcoreMesh(core_axis_name="c", subcore_axis_name="s")
    f = pl.kernel(
        _body,
        out_type=jax.ShapeDtypeStruct((N * H * W, C), jnp.float32),
        mesh=mesh,
        scratch_types=(
            [pltpu.VMEM((64,), jnp.float32)]
            + [pltpu.VMEM((PIECE,), jnp.int32)] * 8
            + [pltpu.VMEM((PIECE,), jnp.float32)] * 4
            + [pltpu.VMEM((PIECE, CP), jnp.float32)] * 8
            + [pltpu.VMEM((PIECE, C), jnp.float32)]
            + [pltpu.SemaphoreType.DMA] * 3
        ),
    )
    return f(tbl, theta_pad)


_PLANES_PER_BLK = 32


def _tab_block(x_ref, o_ref):
    t = jnp.transpose(x_ref[...], (0, 2, 1))
    o_ref[...] = jnp.concatenate(
        [t, jnp.zeros((_PLANES_PER_BLK, W, CP - C), jnp.float32)], axis=2)


def _build_table(xplanar):
    PL = _PLANES_PER_BLK
    tab = pl.pallas_call(
        _tab_block,
        grid=(N * H // PL,),
        in_specs=[pl.BlockSpec((PL, C, W), lambda g: (g, 0, 0))],
        out_specs=pl.BlockSpec((PL, W, CP), lambda g: (g, 0, 0)),
        out_shape=jax.ShapeDtypeStruct((N * H, W, CP), jnp.float32),
    )(xplanar)
    return tab.reshape(N * H * W, CP)


def kernel(X, theta):
    xplanar = jnp.transpose(X, (0, 1, 3, 2)).reshape(N * H, C, W)
    tbl = _build_table(xplanar)
    theta_pad = jnp.concatenate(
        [theta.reshape(-1), jnp.zeros(16, jnp.float32)])
    out = _run(tbl, theta_pad)
    return out.reshape(N, H, W, C)


# speedup vs baseline: 1.9515x; 1.0037x over previous
import functools

import jax
import jax.numpy as jnp
from jax import lax
from jax.experimental import pallas as pl
from jax.experimental.pallas import tpu as pltpu
from jax.experimental.pallas import tpu_sc as plsc

N, H, W, C = 8, 224, 224, 96
NC, NS = 2, 16
NW = NC * NS
ROWS_PER_N = H // NW
PIECE = 64
NPIECE = 4
NCHUNK = PIECE // 16
NPAIR = ROWS_PER_N * NPIECE // 2
CBLK = C // 16
CP = 128

_SCALE = 2.0 / (W - 1)


def _bf16_round(x):
    u = lax.bitcast_convert_type(x, jnp.int32)
    lsb = lax.shift_right_logical(u, 16) & 1
    r = (u + 32767 + lsb) & jnp.int32(-65536)
    return lax.bitcast_convert_type(r, jnp.float32)


def _floor_clip(x):
    t = x.astype(jnp.int32)
    tf = t.astype(jnp.float32)
    fl = jnp.where(tf > x, t - 1, t)
    i0 = jnp.minimum(jnp.maximum(fl, 0), W - 1)
    i1 = jnp.minimum(jnp.maximum(fl + 1, 0), W - 1)
    return i0, i1, i0.astype(jnp.float32), i1.astype(jnp.float32)


def _body(tbl, theta_hbm, out_hbm,
          theta_v,
          ia0, ib0, ic0, id0, ia1, ib1, ic1, id1,
          u0, v0, u1, v1,
          a0, b0, c0, d0, a1, b1, c1, d1,
          out_v, sg0, sg1, so):
    wid = lax.axis_index("s") * NC + lax.axis_index("c")

    pltpu.sync_copy(theta_hbm, theta_v)

    iota = lax.iota(jnp.int32, 16)

    IDX = ((ia0, ib0, ic0, id0), (ia1, ib1, ic1, id1))
    WT = ((u0, v0), (u1, v1))
    BUF = ((a0, b0, c0, d0), (a1, b1, c1, d1))
    SG = (sg0, sg1)

    def chunk_coords(th, yt, jb, pb):
        a00, a01, a02 = th[0], th[1], th[2]
        a10, a11, a12 = th[3], th[4], th[5]
        jv = jb + pb + iota
        xt = _bf16_round(jv.astype(jnp.float32) * _SCALE - 1.0)
        xs = a00 * xt + a01 * yt + a02
        ys = a10 * xt + a11 * yt + a12
        xv = (xs + 1.0) * (W / 2)
        yv = (ys + 1.0) * (H / 2)
        return xv, yv

    def row_consts(t):
        i = wid * ROWS_PER_N + (t >> 2)
        piece = t & 3
        jb = jnp.where(piece == 3, W - PIECE, piece * PIECE)
        yt = _bf16_round((iota * 0 + i).astype(jnp.float32) * _SCALE - 1.0)
        return i, jb, yt

    def coords_fire(th, base_row, t, P):
        ia_v, ib_v, ic_v, id_v = IDX[P]
        u_v, v_v = WT[P]
        i, jb, yt = row_consts(t)
        for k in range(NCHUNK):
            sl = pl.ds(k * 16, 16)
            xv, yv = chunk_coords(th, yt, jb, k * 16)
            x0, x1, x0f, x1f = _floor_clip(xv)
            y0, y1, y0f, y1f = _floor_clip(yv)
            ia_v[sl] = base_row + y0 * W + x0
            ib_v[sl] = base_row + y1 * W + x0
            ic_v[sl] = base_row + y0 * W + x1
            id_v[sl] = base_row + y1 * W + x1
            u_v[sl] = x1f - xv
            v_v[sl] = y1f - yv
        for x in range(4):
            pltpu.async_copy(tbl.at[IDX[P][x]], BUF[P][x], SG[P])

    def drain_out():
        pltpu.make_async_copy(out_v, out_hbm.at[pl.ds(0, PIECE)], so).wait()

    def combine(th, base_row, t, P, first):
        for x in range(4):
            pltpu.make_async_copy(tbl.at[IDX[P][x]], BUF[P][x],
                                  SG[P]).wait()
        if first is None:
            drain_out()
        else:
            @pl.when(jnp.logical_not(first))
            def _():
                drain_out()
        bufa, bufb, bufc, bufd = BUF[P]
        ia_v, ib_v, ic_v, _idv = IDX[P]
        u_v, v_v = WT[P]
        i, jb, yt = row_consts(t)

        def pix_chunk(q, _):
            pb = q * 16
            sl = pl.ds(pb, 16)
            ia16 = ia_v[sl]
            ib16 = ib_v[sl]
            ic16 = ic_v[sl]
            uu16 = u_v[sl]
            vv16 = v_v[sl]
            uup16 = (ic16 - ia16).astype(jnp.float32) - uu16
            vvp16 = jnp.where(ib16 > ia16, 1.0, 0.0) - vv16
            for l in range(16):
                p = pb + l
                uu = uu16[l]
                uup = uup16[l]
                vv = vv16[l]
                vvp = vvp16[l]
                for c in range(CBLK):
                    cs = pl.ds(c * 16, 16)
                    sa = bufa[p, cs]
                    sb = bufb[p, cs]
                    sc = bufc[p, cs]
                    sd = bufd[p, cs]
                    m1 = vv * sa + vvp * sb
                    m2 = vv * sc + vvp * sd
                    out_v[p, cs] = uu * m1 + uup * m2
            return 0

        lax.fori_loop(0, NCHUNK, pix_chunk, 0)

        dst = base_row + i * W + jb
        pltpu.async_copy(out_v, out_hbm.at[pl.ds(dst, PIECE)], so)

    def sample_body(n, _):
        base_row = n * (H * W)
        th = _bf16_round(theta_v[pl.ds(n * 6, 16)])
        coords_fire(th, base_row, 0, 0)

        def pair(s, _):
            coords_fire(th, base_row, 2 * s + 1, 1)
            combine(th, base_row, 2 * s, 0,
                    first=jnp.logical_and(n == 0, s == 0))

            @pl.when(s < NPAIR - 1)
            def _():
                coords_fire(th, base_row, 2 * s + 2, 0)

            combine(th, base_row, 2 * s + 1, 1, first=None)
            return 0

        lax.fori_loop(0, NPAIR, pair, 0)
        return 0

    lax.fori_loop(0, N, sample_body, 0)
    drain_out()


@jax.jit
def _run(tbl, theta_pad):
    mesh = plsc.VectorSubcoreMesh(core_axis_name="c", subcore_axis_name="s")
    f = pl.kernel(
        _body,
        out_type=jax.ShapeDtypeStruct((N * H * W, C), jnp.float32),
        mesh=mesh,
        scratch_types=(
            [pltpu.VMEM((64,), jnp.float32)]
            + [pltpu.VMEM((PIECE,), jnp.int32)] * 8
            + [pltpu.VMEM((PIECE,), jnp.float32)] * 4
            + [pltpu.VMEM((PIECE, CP), jnp.float32)] * 8
            + [pltpu.VMEM((PIECE, C), jnp.float32)]
            + [pltpu.SemaphoreType.DMA] * 3
        ),
    )
    return f(tbl, theta_pad)


_PLANES_PER_BLK = 64


def _tab_block(x_ref, o_ref):
    t = jnp.transpose(x_ref[...], (0, 2, 1))
    o_ref[...] = jnp.concatenate(
        [t, jnp.zeros((_PLANES_PER_BLK, W, CP - C), jnp.float32)], axis=2)


def _build_table(xplanar):
    PL = _PLANES_PER_BLK
    tab = pl.pallas_call(
        _tab_block,
        grid=(N * H // PL,),
        in_specs=[pl.BlockSpec((PL, C, W), lambda g: (g, 0, 0))],
        out_specs=pl.BlockSpec((PL, W, CP), lambda g: (g, 0, 0)),
        out_shape=jax.ShapeDtypeStruct((N * H, W, CP), jnp.float32),
    )(xplanar)
    return tab.reshape(N * H * W, CP)


def kernel(X, theta):
    xplanar = jnp.transpose(X, (0, 1, 3, 2)).reshape(N * H, C, W)
    tbl = _build_table(xplanar)
    theta_pad = jnp.concatenate(
        [theta.reshape(-1), jnp.zeros(16, jnp.float32)])
    out = _run(tbl, theta_pad)
    return out.reshape(N, H, W, C)


# speedup vs baseline: 1.9590x; 1.0038x over previous
import jax
import jax.numpy as jnp
from jax import lax
from jax.experimental import pallas as pl
from jax.experimental.pallas import tpu as pltpu
from jax.experimental.pallas import tpu_sc as plsc

N, H, W, C = 8, 224, 224, 96
NC, NS = 2, 16
NW = NC * NS
ROWS_PER_N = H // NW
PIECE = 64
NPIECE = 4
NCHUNK = PIECE // 16
NPAIR = ROWS_PER_N * NPIECE // 2
CBLK = C // 16
CP = 128

_SCALE = 2.0 / (W - 1)


def _bf16_round(x):
    u = lax.bitcast_convert_type(x, jnp.int32)
    lsb = lax.shift_right_logical(u, 16) & 1
    r = (u + 32767 + lsb) & jnp.int32(-65536)
    return lax.bitcast_convert_type(r, jnp.float32)


def _floor_clip(x):
    t = x.astype(jnp.int32)
    tf = t.astype(jnp.float32)
    fl = jnp.where(tf > x, t - 1, t)
    i0 = jnp.minimum(jnp.maximum(fl, 0), W - 1)
    i1 = jnp.minimum(jnp.maximum(fl + 1, 0), W - 1)
    return i0, i1, i0.astype(jnp.float32), i1.astype(jnp.float32)


def _body(tbl, theta_hbm, out_hbm,
          theta_v,
          ia0, ib0, ic0, id0, ia1, ib1, ic1, id1,
          u0, v0, u1, v1,
          a0, b0, c0, d0, a1, b1, c1, d1,
          out_v, sg0, sg1, so):
    wid = lax.axis_index("s") * NC + lax.axis_index("c")

    pltpu.sync_copy(theta_hbm, theta_v)

    iota = lax.iota(jnp.int32, 16)

    IDX = ((ia0, ib0, ic0, id0), (ia1, ib1, ic1, id1))
    WT = ((u0, v0), (u1, v1))
    BUF = ((a0, b0, c0, d0), (a1, b1, c1, d1))
    SG = (sg0, sg1)

    def chunk_coords(th, yt, jb, pb):
        a00, a01, a02 = th[0], th[1], th[2]
        a10, a11, a12 = th[3], th[4], th[5]
        jv = jb + pb + iota
        xt = _bf16_round(jv.astype(jnp.float32) * _SCALE - 1.0)
        xs = a00 * xt + a01 * yt + a02
        ys = a10 * xt + a11 * yt + a12
        xv = (xs + 1.0) * (W / 2)
        yv = (ys + 1.0) * (H / 2)
        return xv, yv

    def row_consts(t):
        i = wid * ROWS_PER_N + (t >> 2)
        piece = t & 3
        jb = jnp.where(piece == 3, W - PIECE, piece * PIECE)
        yt = _bf16_round((iota * 0 + i).astype(jnp.float32) * _SCALE - 1.0)
        return i, jb, yt

    def coords_fire(th, base_row, t, P):
        ia_v, ib_v, ic_v, id_v = IDX[P]
        u_v, v_v = WT[P]
        i, jb, yt = row_consts(t)
        for k in range(NCHUNK):
            sl = pl.ds(k * 16, 16)
            xv, yv = chunk_coords(th, yt, jb, k * 16)
            x0, x1, x0f, x1f = _floor_clip(xv)
            y0, y1, y0f, y1f = _floor_clip(yv)
            ia_v[sl] = base_row + y0 * W + x0
            ib_v[sl] = base_row + y1 * W + x0
            ic_v[sl] = base_row + y0 * W + x1
            id_v[sl] = base_row + y1 * W + x1
            u_v[sl] = x1f - xv
            v_v[sl] = y1f - yv
        for x in range(4):
            pltpu.async_copy(tbl.at[IDX[P][x]], BUF[P][x], SG[P])

    def drain_out():
        pltpu.make_async_copy(out_v, out_hbm.at[pl.ds(0, PIECE)], so).wait()

    def combine(th, base_row, t, P, first):
        for x in range(4):
            pltpu.make_async_copy(tbl.at[IDX[P][x]], BUF[P][x],
                                  SG[P]).wait()
        if first is None:
            drain_out()
        else:
            @pl.when(jnp.logical_not(first))
            def _():
                drain_out()
        bufa, bufb, bufc, bufd = BUF[P]
        ia_v, ib_v, ic_v, _idv = IDX[P]
        u_v, v_v = WT[P]
        i, jb, yt = row_consts(t)

        def pix_chunk(q, _):
            pb = q * 16
            sl = pl.ds(pb, 16)
            ia16 = ia_v[sl]
            ib16 = ib_v[sl]
            ic16 = ic_v[sl]
            uu16 = u_v[sl]
            vv16 = v_v[sl]
            uup16 = (ic16 - ia16).astype(jnp.float32) - uu16
            vvp16 = jnp.where(ib16 > ia16, 1.0, 0.0) - vv16
            for l in range(16):
                p = pb + l
                uu = uu16[l]
                uup = uup16[l]
                vv = vv16[l]
                vvp = vvp16[l]
                for c in range(CBLK):
                    cs = pl.ds(c * 16, 16)
                    sa = bufa[p, cs]
                    sb = bufb[p, cs]
                    sc = bufc[p, cs]
                    sd = bufd[p, cs]
                    m1 = vv * sa + vvp * sb
                    m2 = vv * sc + vvp * sd
                    out_v[p, cs] = uu * m1 + uup * m2
            return 0

        lax.fori_loop(0, NCHUNK, pix_chunk, 0)

        dst = base_row + i * W + jb
        pltpu.async_copy(out_v, out_hbm.at[pl.ds(dst, PIECE)], so)

    def sample_body(n, _):
        base_row = n * (H * W)
        th = _bf16_round(theta_v[pl.ds(n * 6, 16)])
        coords_fire(th, base_row, 0, 0)

        def pair(s, _):
            coords_fire(th, base_row, 2 * s + 1, 1)
            combine(th, base_row, 2 * s, 0,
                    first=jnp.logical_and(n == 0, s == 0))

            @pl.when(s < NPAIR - 1)
            def _():
                coords_fire(th, base_row, 2 * s + 2, 0)

            combine(th, base_row, 2 * s + 1, 1, first=None)
            return 0

        lax.fori_loop(0, NPAIR, pair, 0)
        return 0

    lax.fori_loop(0, N, sample_body, 0)
    drain_out()


@jax.jit
def _run(tbl, theta_pad):
    mesh = plsc.VectorSubcoreMesh(core_axis_name="c", subcore_axis_name="s")
    f = pl.kernel(
        _body,
        out_type=jax.ShapeDtypeStruct((N * H * W, C), jnp.float32),
        mesh=mesh,
        scratch_types=(
            [pltpu.VMEM((64,), jnp.float32)]
            + [pltpu.VMEM((PIECE,), jnp.int32)] * 8
            + [pltpu.VMEM((PIECE,), jnp.float32)] * 4
            + [pltpu.VMEM((PIECE, CP), jnp.float32)] * 8
            + [pltpu.VMEM((PIECE, C), jnp.float32)]
            + [pltpu.SemaphoreType.DMA] * 3
        ),
    )
    return f(tbl, theta_pad)


_PLANES_PER_BLK = 64


def _tab_block(x_ref, o_ref):
    t = jnp.transpose(x_ref[...], (0, 2, 1))
    o_ref[...] = jnp.concatenate(
        [t, jnp.zeros((_PLANES_PER_BLK, W, CP - C), jnp.float32)], axis=2)


def _build_table(xplanar):
    PL = _PLANES_PER_BLK
    tab = pl.pallas_call(
        _tab_block,
        grid=(N * H // PL,),
        in_specs=[pl.BlockSpec((PL, C, W), lambda g: (g, 0, 0))],
        out_specs=pl.BlockSpec((PL, W, CP), lambda g: (g, 0, 0)),
        out_shape=jax.ShapeDtypeStruct((N * H, W, CP), jnp.float32),
    )(xplanar)
    return tab.reshape(N * H * W, CP)


def kernel(X, theta):
    xplanar = jnp.transpose(X, (0, 1, 3, 2)).reshape(N * H, C, W)
    tbl = _build_table(xplanar)
    theta_pad = jnp.concatenate(
        [theta.reshape(-1), jnp.zeros(16, jnp.float32)])
    out = _run(tbl, theta_pad)
    return out.reshape(N, H, W, C)
